# baseline (speedup 1.0000x reference)
import functools

import jax
import jax.numpy as jnp
from jax import lax
from jax.experimental import pallas as pl
from jax.experimental.pallas import tpu as pltpu
from jax.experimental.pallas import tpu_sc as plsc

NC = 2
NS = 16
NW = NC * NS
LANES = 16
D = 128


def _mesh():
    return plsc.VectorSubcoreMesh(
        core_axis_name="c", subcore_axis_name="s", num_cores=NC, num_subcores=NS
    )


def _make_deg_kernel(npad, nch, ch):
    rpt = npad // NS

    @functools.partial(
        pl.kernel,
        out_type=jax.ShapeDtypeStruct((NC * npad,), jnp.float32),
        mesh=_mesh(),
        scratch_types=[
            pltpu.VMEM((nch, ch), jnp.int32),
            pltpu.VMEM((ch,), jnp.float32),
            pltpu.VMEM((rpt,), jnp.float32),
            pltpu.VMEM_SHARED((npad,), jnp.float32),
        ],
    )
    def deg_kernel(dst_hbm, out_hbm, idx_v, ones_v, zb_v, deg_sp):
        c = lax.axis_index("c")
        s = lax.axis_index("s")
        wid = s * NC + c

        @pl.loop(0, ch // LANES)
        def _(i):
            ones_v[pl.ds(i * LANES, LANES)] = jnp.ones((LANES,), jnp.float32)

        @pl.loop(0, rpt // LANES)
        def _(i):
            zb_v[pl.ds(i * LANES, LANES)] = jnp.zeros((LANES,), jnp.float32)

        pltpu.sync_copy(zb_v, deg_sp.at[pl.ds(s * rpt, rpt)])
        plsc.subcore_barrier()

        pltpu.sync_copy(dst_hbm.at[wid], idx_v)

        @pl.loop(0, nch)
        def _(j):
            pltpu.sync_copy(ones_v, deg_sp.at[idx_v.at[j]], add=True)

        plsc.subcore_barrier()
        pltpu.sync_copy(
            deg_sp.at[pl.ds(s * rpt, rpt)],
            out_hbm.at[pl.ds(c * npad + s * rpt, rpt)],
        )

    return deg_kernel


def _make_agg_kernel(npad2, nch, ch):
    rpt = npad2 // NS
    nz = rpt // ch
    ntrash = 64
    acc_rows = npad2 + ntrash

    @functools.partial(
        pl.kernel,
        out_type=jax.ShapeDtypeStruct((NC, npad2, D), jnp.float32),
        mesh=_mesh(),
        scratch_types=[
            pltpu.VMEM((nch, ch), jnp.int32),
            pltpu.VMEM((nch, ch), jnp.int32),
            pltpu.VMEM((ch, D), jnp.float32),
            pltpu.VMEM((ch, D), jnp.float32),
            pltpu.VMEM_SHARED((acc_rows, D), jnp.float32),
            pltpu.SemaphoreType.DMA,
            pltpu.SemaphoreType.DMA,
            pltpu.SemaphoreType.DMA,
            pltpu.SemaphoreType.DMA,
        ],
    )
    def agg_kernel(
        g_hbm, src_hbm, dst_hbm, out_hbm,
        src_v, dst_v, rows0_v, rows1_v, s_sp, gsem0, gsem1, ssem0, ssem1,
    ):
        c = lax.axis_index("c")
        s = lax.axis_index("s")
        base = c * npad2

        @pl.loop(0, ch)
        def _(r):
            @pl.loop(0, D // LANES)
            def _(i):
                rows0_v[pl.ds(r, 1), pl.ds(i * LANES, LANES)] = jnp.zeros(
                    (1, LANES), jnp.float32
                )

        @pl.loop(0, nz)
        def _(k):
            pltpu.sync_copy(rows0_v, s_sp.at[pl.ds(s * rpt + k * ch, ch)])

        rem = rpt - nz * ch
        if rem:
            pltpu.sync_copy(
                rows0_v.at[pl.ds(0, rem)],
                s_sp.at[pl.ds(s * rpt + nz * ch, rem)],
            )

        plsc.subcore_barrier()

        pltpu.sync_copy(src_hbm.at[s], src_v)
        pltpu.sync_copy(dst_hbm.at[s], dst_v)

        @pl.loop(0, nch)
        def _(j):
            @pl.loop(0, ch // LANES)
            def _(k):
                v = dst_v[j, pl.ds(k * LANES, LANES)]
                ok = (v >= base) & (v < base + npad2)
                tr = npad2 + (v & (ntrash - 1))
                dst_v[j, pl.ds(k * LANES, LANES)] = jnp.where(ok, v - base, tr)

        def gather(j, buf, sem):
            pltpu.async_copy(g_hbm.at[src_v.at[j]], buf, sem)

        def drain_g(buf, sem):
            pltpu.make_async_copy(g_hbm.at[pl.ds(0, ch)], buf, sem).wait()

        def scatter(j, buf, sem):
            pltpu.async_copy(buf, s_sp.at[dst_v.at[j]], sem, add=True)

        def drain_s(j, buf, sem):
            pltpu.make_async_copy(buf, s_sp.at[dst_v.at[j]], sem).wait()

        gather(0, rows0_v, gsem0)
        gather(1, rows1_v, gsem1)

        @pl.loop(0, nch // 2 - 1)
        def _(jj):
            j0 = jj * 2
            drain_g(rows0_v, gsem0)
            scatter(j0, rows0_v, ssem0)
            drain_g(rows1_v, gsem1)
            scatter(j0 + 1, rows1_v, ssem1)
            drain_s(j0, rows0_v, ssem0)
            gather(j0 + 2, rows0_v, gsem0)
            drain_s(j0 + 1, rows1_v, ssem1)
            gather(j0 + 3, rows1_v, gsem1)

        drain_g(rows0_v, gsem0)
        scatter(nch - 2, rows0_v, ssem0)
        drain_g(rows1_v, gsem1)
        scatter(nch - 1, rows1_v, ssem1)
        drain_s(nch - 2, rows0_v, ssem0)
        drain_s(nch - 1, rows1_v, ssem1)

        plsc.subcore_barrier()
        pltpu.sync_copy(
            s_sp.at[pl.ds(s * rpt, rpt)], out_hbm.at[c, pl.ds(s * rpt, rpt)]
        )

    return agg_kernel


def _mm_scale_body(x_ref, w_ref, deg_ref, g_ref, dinv_ref):
    deg = deg_ref[0] + deg_ref[1] + 1.0
    dinv = lax.rsqrt(deg)
    h = jnp.dot(x_ref[...], w_ref[...], preferred_element_type=jnp.float32)
    g_ref[...] = h * dinv
    dinv_ref[...] = dinv


def _mid_body(s_ref, g_ref, dinv_ref, b_ref, w_ref, g2_ref):
    dinv = dinv_ref[...]
    t = (s_ref[0] + g_ref[...]) * dinv + b_ref[...]
    z = jnp.maximum(t, 0.0)
    g2_ref[...] = jnp.dot(z, w_ref[...], preferred_element_type=jnp.float32) * dinv


def _final_body(s_ref, g_ref, dinv_ref, b_ref, out_ref):
    out_ref[...] = (s_ref[0] + g_ref[...]) * dinv_ref[...] + b_ref[...]


@jax.jit
def kernel(x, edge_index, W1, b1, W2, b2):
    n, d = x.shape
    e = edge_index.shape[1]
    npad = ((n + NS * 128 - 1) // (NS * 128)) * (NS * 128)
    npad2 = npad // NC
    chd = 80
    nchd = e // (NW * chd)
    ch = 128
    nch = -(-e // (NS * ch))
    nch += nch % 2
    epad = NS * nch * ch - e

    src_i = edge_index[0].astype(jnp.int32)
    dst_i = edge_index[1].astype(jnp.int32)
    dst_d = dst_i.reshape(NW, nchd, chd)
    pad_src = jnp.arange(epad, dtype=jnp.int32) % jnp.int32(n)
    pad_dst = jnp.full((epad,), npad, jnp.int32)
    src_a = jnp.concatenate([src_i, pad_src]).reshape(NS, nch, ch)
    dst_a = jnp.concatenate([dst_i, pad_dst]).reshape(NS, nch, ch)
    x_pad = jnp.zeros((npad, d), x.dtype).at[:n].set(x)
    b1r = b1.reshape(1, D)
    b2r = b2.reshape(1, D)

    deg = _make_deg_kernel(npad, nchd, chd)(dst_d)
    deg3 = deg.reshape(NC, npad, 1)

    bm = 1024
    npb = npad2 // bm
    grid = (npad // bm,)
    row = lambda i: (i, 0)
    s_spec = pl.BlockSpec((1, bm, D), lambda i: (i // npb, i % npb, 0))
    x_spec = pl.BlockSpec((bm, D), row)
    w_spec = pl.BlockSpec((D, D), lambda i: (0, 0))
    v_spec = pl.BlockSpec((bm, 1), row)
    b_spec = pl.BlockSpec((1, D), lambda i: (0, 0))
    deg_spec = pl.BlockSpec((NC, bm, 1), lambda i: (0, i, 0))

    g1, dinv = pl.pallas_call(
        _mm_scale_body,
        grid=grid,
        in_specs=[x_spec, w_spec, deg_spec],
        out_specs=[x_spec, v_spec],
        out_shape=[
            jax.ShapeDtypeStruct((npad, D), jnp.float32),
            jax.ShapeDtypeStruct((npad, 1), jnp.float32),
        ],
    )(x_pad, W1, deg3)

    agg = _make_agg_kernel(npad2, nch, ch)
    s1 = agg(g1, src_a, dst_a)

    g2 = pl.pallas_call(
        _mid_body,
        grid=grid,
        in_specs=[s_spec, x_spec, v_spec, b_spec, w_spec],
        out_specs=x_spec,
        out_shape=jax.ShapeDtypeStruct((npad, D), jnp.float32),
    )(s1, g1, dinv, b1r, W2)

    s2 = agg(g2, src_a, dst_a)

    out = pl.pallas_call(
        _final_body,
        grid=grid,
        in_specs=[s_spec, x_spec, v_spec, b_spec],
        out_specs=x_spec,
        out_shape=jax.ShapeDtypeStruct((npad, D), jnp.float32),
    )(s2, g2, dinv, b2r)

    return out[:n]


# speedup vs baseline: 1.0942x; 1.0942x over previous
import functools

import jax
import jax.numpy as jnp
from jax import lax
from jax.experimental import pallas as pl
from jax.experimental.pallas import tpu as pltpu
from jax.experimental.pallas import tpu_sc as plsc

NC = 2
NS = 16
NW = NC * NS
LANES = 16
D = 128


def _mesh():
    return plsc.VectorSubcoreMesh(
        core_axis_name="c", subcore_axis_name="s", num_cores=NC, num_subcores=NS
    )


def _make_deg_kernel(npad, nch, ch):
    rpt = npad // NS

    @functools.partial(
        pl.kernel,
        out_type=jax.ShapeDtypeStruct((NC * npad,), jnp.float32),
        mesh=_mesh(),
        scratch_types=[
            pltpu.VMEM((nch, ch), jnp.int32),
            pltpu.VMEM((ch,), jnp.float32),
            pltpu.VMEM((rpt,), jnp.float32),
            pltpu.VMEM_SHARED((npad,), jnp.float32),
        ],
    )
    def deg_kernel(dst_hbm, out_hbm, idx_v, ones_v, zb_v, deg_sp):
        c = lax.axis_index("c")
        s = lax.axis_index("s")
        wid = s * NC + c

        @pl.loop(0, ch // LANES)
        def _(i):
            ones_v[pl.ds(i * LANES, LANES)] = jnp.ones((LANES,), jnp.float32)

        @pl.loop(0, rpt // LANES)
        def _(i):
            zb_v[pl.ds(i * LANES, LANES)] = jnp.zeros((LANES,), jnp.float32)

        pltpu.sync_copy(zb_v, deg_sp.at[pl.ds(s * rpt, rpt)])
        plsc.subcore_barrier()

        pltpu.sync_copy(dst_hbm.at[wid], idx_v)

        @pl.loop(0, nch)
        def _(j):
            pltpu.sync_copy(ones_v, deg_sp.at[idx_v.at[j]], add=True)

        plsc.subcore_barrier()
        pltpu.sync_copy(
            deg_sp.at[pl.ds(s * rpt, rpt)],
            out_hbm.at[pl.ds(c * npad + s * rpt, rpt)],
        )

    return deg_kernel


def _make_agg_kernel(npad2, nch, ch):
    rpt = npad2 // NS
    nz = rpt // ch
    ntrash = 64
    acc_rows = npad2 + ntrash

    @functools.partial(
        pl.kernel,
        out_type=jax.ShapeDtypeStruct((NC, npad2, D), jnp.float32),
        mesh=_mesh(),
        scratch_types=[
            pltpu.VMEM((nch, ch), jnp.int32),
            pltpu.VMEM((nch, ch), jnp.int32),
            pltpu.VMEM((ch, D), jnp.float32),
            pltpu.VMEM((ch, D), jnp.float32),
            pltpu.VMEM_SHARED((acc_rows, D), jnp.float32),
            pltpu.SemaphoreType.DMA,
            pltpu.SemaphoreType.DMA,
        ],
    )
    def agg_kernel(
        g_hbm, src_hbm, dst_hbm, out_hbm,
        src_v, dst_v, rows0_v, rows1_v, s_sp, gsem0, gsem1,
    ):
        c = lax.axis_index("c")
        s = lax.axis_index("s")
        base = c * npad2

        @pl.loop(0, ch)
        def _(r):
            @pl.loop(0, D // LANES)
            def _(i):
                rows0_v[pl.ds(r, 1), pl.ds(i * LANES, LANES)] = jnp.zeros(
                    (1, LANES), jnp.float32
                )

        @pl.loop(0, nz)
        def _(k):
            pltpu.sync_copy(rows0_v, s_sp.at[pl.ds(s * rpt + k * ch, ch)])

        rem = rpt - nz * ch
        if rem:
            pltpu.sync_copy(
                rows0_v.at[pl.ds(0, rem)],
                s_sp.at[pl.ds(s * rpt + nz * ch, rem)],
            )

        plsc.subcore_barrier()

        pltpu.sync_copy(src_hbm.at[s], src_v)
        pltpu.sync_copy(dst_hbm.at[s], dst_v)

        @pl.loop(0, nch)
        def _(j):
            @pl.loop(0, ch // LANES)
            def _(k):
                v = dst_v[j, pl.ds(k * LANES, LANES)]
                ok = (v >= base) & (v < base + npad2)
                tr = npad2 + (v & (ntrash - 1))
                dst_v[j, pl.ds(k * LANES, LANES)] = jnp.where(ok, v - base, tr)

        def gather(j, buf, sem):
            pltpu.async_copy(g_hbm.at[src_v.at[j]], buf, sem)

        def drain_g(buf, sem):
            pltpu.make_async_copy(g_hbm.at[pl.ds(0, ch)], buf, sem).wait()

        def scatter(j, buf):
            pltpu.sync_copy(buf, s_sp.at[dst_v.at[j]], add=True)

        gather(0, rows0_v, gsem0)

        @pl.loop(0, nch // 2 - 1)
        def _(jj):
            j0 = jj * 2
            drain_g(rows0_v, gsem0)
            gather(j0 + 1, rows1_v, gsem1)
            scatter(j0, rows0_v)
            drain_g(rows1_v, gsem1)
            gather(j0 + 2, rows0_v, gsem0)
            scatter(j0 + 1, rows1_v)

        drain_g(rows0_v, gsem0)
        gather(nch - 1, rows1_v, gsem1)
        scatter(nch - 2, rows0_v)
        drain_g(rows1_v, gsem1)
        scatter(nch - 1, rows1_v)

        plsc.subcore_barrier()
        pltpu.sync_copy(
            s_sp.at[pl.ds(s * rpt, rpt)], out_hbm.at[c, pl.ds(s * rpt, rpt)]
        )

    return agg_kernel


def _mm_body(x_ref, w_ref, h_ref):
    h_ref[...] = jnp.dot(x_ref[...], w_ref[...], preferred_element_type=jnp.float32)


def _scale_body(h_ref, deg_ref, g_ref, dinv_ref):
    deg = deg_ref[0] + deg_ref[1] + 1.0
    dinv = lax.rsqrt(deg)
    g_ref[...] = h_ref[...] * dinv
    dinv_ref[...] = dinv


def _mid_body(s_ref, g_ref, dinv_ref, b_ref, w_ref, g2_ref):
    dinv = dinv_ref[...]
    t = (s_ref[0] + g_ref[...]) * dinv + b_ref[...]
    z = jnp.maximum(t, 0.0)
    g2_ref[...] = jnp.dot(z, w_ref[...], preferred_element_type=jnp.float32) * dinv


def _final_body(s_ref, g_ref, dinv_ref, b_ref, out_ref):
    out_ref[...] = (s_ref[0] + g_ref[...]) * dinv_ref[...] + b_ref[...]


@jax.jit
def kernel(x, edge_index, W1, b1, W2, b2):
    n, d = x.shape
    e = edge_index.shape[1]
    npad = ((n + NS * 128 - 1) // (NS * 128)) * (NS * 128)
    npad2 = npad // NC
    chd = 80
    nchd = e // (NW * chd)
    ch = 128
    ept = e // NS
    nch = -(-ept // ch)
    nch += nch % 2
    tpad = nch * ch - ept

    src_i = edge_index[0].astype(jnp.int32)
    dst_i = edge_index[1].astype(jnp.int32)
    dst_d = dst_i.reshape(NW, nchd, chd)
    pad_src = jnp.tile(jnp.arange(tpad, dtype=jnp.int32)[None, :] * 13 % jnp.int32(n), (NS, 1))
    pad_dst = jnp.full((NS, tpad), npad, jnp.int32)
    src_a = jnp.concatenate([src_i.reshape(NS, ept), pad_src], 1).reshape(NS, nch, ch)
    dst_a = jnp.concatenate([dst_i.reshape(NS, ept), pad_dst], 1).reshape(NS, nch, ch)
    x_pad = jnp.zeros((npad, d), x.dtype).at[:n].set(x)
    b1r = b1.reshape(1, D)
    b2r = b2.reshape(1, D)

    deg = _make_deg_kernel(npad, nchd, chd)(dst_d)
    deg3 = deg.reshape(NC, npad, 1)

    bm = 1024
    npb = npad2 // bm
    grid = (npad // bm,)
    row = lambda i: (i, 0)
    s_spec = pl.BlockSpec((1, bm, D), lambda i: (i // npb, i % npb, 0))
    x_spec = pl.BlockSpec((bm, D), row)
    w_spec = pl.BlockSpec((D, D), lambda i: (0, 0))
    v_spec = pl.BlockSpec((bm, 1), row)
    b_spec = pl.BlockSpec((1, D), lambda i: (0, 0))
    deg_spec = pl.BlockSpec((NC, bm, 1), lambda i: (0, i, 0))

    h1 = pl.pallas_call(
        _mm_body,
        grid=grid,
        in_specs=[x_spec, w_spec],
        out_specs=x_spec,
        out_shape=jax.ShapeDtypeStruct((npad, D), jnp.float32),
    )(x_pad, W1)

    g1, dinv = pl.pallas_call(
        _scale_body,
        grid=grid,
        in_specs=[x_spec, deg_spec],
        out_specs=[x_spec, v_spec],
        out_shape=[
            jax.ShapeDtypeStruct((npad, D), jnp.float32),
            jax.ShapeDtypeStruct((npad, 1), jnp.float32),
        ],
    )(h1, deg3)

    agg = _make_agg_kernel(npad2, nch, ch)
    s1 = agg(g1, src_a, dst_a)

    g2 = pl.pallas_call(
        _mid_body,
        grid=grid,
        in_specs=[s_spec, x_spec, v_spec, b_spec, w_spec],
        out_specs=x_spec,
        out_shape=jax.ShapeDtypeStruct((npad, D), jnp.float32),
    )(s1, g1, dinv, b1r, W2)

    s2 = agg(g2, src_a, dst_a)

    out = pl.pallas_call(
        _final_body,
        grid=grid,
        in_specs=[s_spec, x_spec, v_spec, b_spec],
        out_specs=x_spec,
        out_shape=jax.ShapeDtypeStruct((npad, D), jnp.float32),
    )(s2, g2, dinv, b2r)

    return out[:n]


# speedup vs baseline: 1.1054x; 1.0103x over previous
import functools

import jax
import jax.numpy as jnp
from jax import lax
from jax.experimental import pallas as pl
from jax.experimental.pallas import tpu as pltpu
from jax.experimental.pallas import tpu_sc as plsc

NC = 2
NS = 16
NW = NC * NS
LANES = 16
D = 128


def _mesh():
    return plsc.VectorSubcoreMesh(
        core_axis_name="c", subcore_axis_name="s", num_cores=NC, num_subcores=NS
    )


def _make_deg_kernel(npad, nch, ch):
    rpt = npad // NS

    @functools.partial(
        pl.kernel,
        out_type=jax.ShapeDtypeStruct((NC * npad,), jnp.float32),
        mesh=_mesh(),
        scratch_types=[
            pltpu.VMEM((nch, ch), jnp.int32),
            pltpu.VMEM((ch,), jnp.float32),
            pltpu.VMEM((rpt,), jnp.float32),
            pltpu.VMEM_SHARED((npad,), jnp.float32),
        ],
    )
    def deg_kernel(dst_hbm, out_hbm, idx_v, ones_v, zb_v, deg_sp):
        c = lax.axis_index("c")
        s = lax.axis_index("s")
        wid = s * NC + c

        @pl.loop(0, ch // LANES)
        def _(i):
            ones_v[pl.ds(i * LANES, LANES)] = jnp.ones((LANES,), jnp.float32)

        @pl.loop(0, rpt // LANES)
        def _(i):
            zb_v[pl.ds(i * LANES, LANES)] = jnp.zeros((LANES,), jnp.float32)

        pltpu.sync_copy(zb_v, deg_sp.at[pl.ds(s * rpt, rpt)])
        plsc.subcore_barrier()

        pltpu.sync_copy(dst_hbm.at[wid], idx_v)

        @pl.loop(0, nch)
        def _(j):
            pltpu.sync_copy(ones_v, deg_sp.at[idx_v.at[j]], add=True)

        plsc.subcore_barrier()
        pltpu.sync_copy(
            deg_sp.at[pl.ds(s * rpt, rpt)],
            out_hbm.at[pl.ds(c * npad + s * rpt, rpt)],
        )

    return deg_kernel


def _make_agg_kernel(npad2, nch, ch):
    rpt = npad2 // NS
    nz = rpt // ch
    ntrash = 64
    acc_rows = npad2 + ntrash

    @functools.partial(
        pl.kernel,
        out_type=jax.ShapeDtypeStruct((NC, npad2, D), jnp.float32),
        mesh=_mesh(),
        scratch_types=[
            pltpu.VMEM((nch, ch), jnp.int32),
            pltpu.VMEM((nch, ch), jnp.int32),
            pltpu.VMEM((ch, D), jnp.float32),
            pltpu.VMEM((ch, D), jnp.float32),
            pltpu.VMEM_SHARED((acc_rows, D), jnp.float32),
            pltpu.SemaphoreType.DMA,
            pltpu.SemaphoreType.DMA,
        ],
    )
    def agg_kernel(
        g_hbm, src_hbm, dst_hbm, out_hbm,
        src_v, dst_v, rows0_v, rows1_v, s_sp, gsem0, gsem1,
    ):
        c = lax.axis_index("c")
        s = lax.axis_index("s")
        base = c * npad2

        @pl.loop(0, ch)
        def _(r):
            @pl.loop(0, D // LANES)
            def _(i):
                rows0_v[pl.ds(r, 1), pl.ds(i * LANES, LANES)] = jnp.zeros(
                    (1, LANES), jnp.float32
                )

        @pl.loop(0, nz)
        def _(k):
            pltpu.sync_copy(rows0_v, s_sp.at[pl.ds(s * rpt + k * ch, ch)])

        rem = rpt - nz * ch
        if rem:
            pltpu.sync_copy(
                rows0_v.at[pl.ds(0, rem)],
                s_sp.at[pl.ds(s * rpt + nz * ch, rem)],
            )

        plsc.subcore_barrier()

        pltpu.sync_copy(src_hbm.at[s], src_v)
        pltpu.sync_copy(dst_hbm.at[s], dst_v)

        @pl.loop(0, nch)
        def _(j):
            @pl.loop(0, ch // LANES)
            def _(k):
                v = dst_v[j, pl.ds(k * LANES, LANES)]
                ok = (v >= base) & (v < base + npad2)
                tr = npad2 + (v & (ntrash - 1))
                dst_v[j, pl.ds(k * LANES, LANES)] = jnp.where(ok, v - base, tr)

        def gather(j, buf, sem):
            pltpu.async_copy(g_hbm.at[src_v.at[j]], buf, sem)

        def drain_g(buf, sem):
            pltpu.make_async_copy(g_hbm.at[pl.ds(0, ch)], buf, sem).wait()

        def scatter(j, buf):
            pltpu.sync_copy(buf, s_sp.at[dst_v.at[j]], add=True)

        gather(0, rows0_v, gsem0)

        @pl.loop(0, nch // 2 - 1)
        def _(jj):
            j0 = jj * 2
            drain_g(rows0_v, gsem0)
            gather(j0 + 1, rows1_v, gsem1)
            scatter(j0, rows0_v)
            drain_g(rows1_v, gsem1)
            gather(j0 + 2, rows0_v, gsem0)
            scatter(j0 + 1, rows1_v)

        drain_g(rows0_v, gsem0)
        gather(nch - 1, rows1_v, gsem1)
        scatter(nch - 2, rows0_v)
        drain_g(rows1_v, gsem1)
        scatter(nch - 1, rows1_v)

        plsc.subcore_barrier()
        pltpu.sync_copy(
            s_sp.at[pl.ds(s * rpt, rpt)], out_hbm.at[c, pl.ds(s * rpt, rpt)]
        )

    return agg_kernel


def _mm_scale_body(x_ref, w_ref, deg_ref, g_ref, dinv_ref):
    deg = deg_ref[0] + deg_ref[1] + 1.0
    dinv = lax.rsqrt(deg)
    h = jnp.dot(x_ref[...], w_ref[...], preferred_element_type=jnp.float32)
    g_ref[...] = h * dinv
    dinv_ref[...] = dinv


def _mid_body(s_ref, g_ref, dinv_ref, b_ref, w_ref, g2_ref):
    dinv = dinv_ref[...]
    t = (s_ref[0] + g_ref[...]) * dinv + b_ref[...]
    z = jnp.maximum(t, 0.0)
    g2_ref[...] = jnp.dot(z, w_ref[...], preferred_element_type=jnp.float32) * dinv


def _final_body(s_ref, g_ref, dinv_ref, b_ref, out_ref):
    out_ref[...] = (s_ref[0] + g_ref[...]) * dinv_ref[...] + b_ref[...]


@jax.jit
def kernel(x, edge_index, W1, b1, W2, b2):
    n, d = x.shape
    e = edge_index.shape[1]
    npad = ((n + NS * 128 - 1) // (NS * 128)) * (NS * 128)
    npad2 = npad // NC
    chd = 80
    nchd = e // (NW * chd)
    ch = 128
    ept = e // NS
    nch = -(-ept // ch)
    nch += nch % 2
    tpad = nch * ch - ept

    src_i = edge_index[0].astype(jnp.int32)
    dst_i = edge_index[1].astype(jnp.int32)
    dst_d = dst_i.reshape(NW, nchd, chd)
    pad_src = jnp.tile(jnp.arange(tpad, dtype=jnp.int32)[None, :] * 13 % jnp.int32(n), (NS, 1))
    pad_dst = jnp.full((NS, tpad), npad, jnp.int32)
    src_a = jnp.concatenate([src_i.reshape(NS, ept), pad_src], 1).reshape(NS, nch, ch)
    dst_a = jnp.concatenate([dst_i.reshape(NS, ept), pad_dst], 1).reshape(NS, nch, ch)
    x_pad = jnp.zeros((npad, d), x.dtype).at[:n].set(x)
    b1r = b1.reshape(1, D)
    b2r = b2.reshape(1, D)

    deg = _make_deg_kernel(npad, nchd, chd)(dst_d)
    deg3 = deg.reshape(NC, npad, 1)

    bm = 1024
    npb = npad2 // bm
    grid = (npad // bm,)
    row = lambda i: (i, 0)
    s_spec = pl.BlockSpec((1, bm, D), lambda i: (i // npb, i % npb, 0))
    x_spec = pl.BlockSpec((bm, D), row)
    w_spec = pl.BlockSpec((D, D), lambda i: (0, 0))
    v_spec = pl.BlockSpec((bm, 1), row)
    b_spec = pl.BlockSpec((1, D), lambda i: (0, 0))
    deg_spec = pl.BlockSpec((NC, bm, 1), lambda i: (0, i, 0))

    g1, dinv = pl.pallas_call(
        _mm_scale_body,
        grid=grid,
        in_specs=[x_spec, w_spec, deg_spec],
        out_specs=[x_spec, v_spec],
        out_shape=[
            jax.ShapeDtypeStruct((npad, D), jnp.float32),
            jax.ShapeDtypeStruct((npad, 1), jnp.float32),
        ],
    )(x_pad, W1, deg3)

    agg = _make_agg_kernel(npad2, nch, ch)
    s1 = agg(g1, src_a, dst_a)

    g2 = pl.pallas_call(
        _mid_body,
        grid=grid,
        in_specs=[s_spec, x_spec, v_spec, b_spec, w_spec],
        out_specs=x_spec,
        out_shape=jax.ShapeDtypeStruct((npad, D), jnp.float32),
    )(s1, g1, dinv, b1r, W2)

    s2 = agg(g2, src_a, dst_a)

    out = pl.pallas_call(
        _final_body,
        grid=grid,
        in_specs=[s_spec, x_spec, v_spec, b_spec],
        out_specs=x_spec,
        out_shape=jax.ShapeDtypeStruct((npad, D), jnp.float32),
    )(s2, g2, dinv, b2r)

    return out[:n]
